# baseline (speedup 1.0000x reference)
import functools

import jax
import jax.numpy as jnp
from jax.experimental import pallas as pl
from jax.experimental.pallas import tpu as pltpu
from jax.experimental.pallas import tpu_sc as plsc

T, N, F = 3, 4096, 128
BR = 512
BM = 512
L = 16
K8 = F // L


def _score_body(xt_ref, p_ref, z_ref):
    z_ref[0] = jax.lax.dot_general(
        p_ref[...], xt_ref[0], (((1,), (0,)), ((), ())),
        preferred_element_type=jnp.float32)


def _prep_body(a_ref, x_ref, u_ref, di_ref, ab_ref):
    a = a_ref[0]
    s = jnp.sum(a, axis=1, keepdims=True) + 1.0
    di = jnp.where(s > 0, jax.lax.rsqrt(s), 0.0)
    u_ref[0] = di * x_ref[0]
    di_ref[0] = di
    ab_ref[0] = a.astype(jnp.bfloat16)


def _chain_body(xh_ref, zt_ref, p_ref, wih_ref, whh_ref, bih_ref, bhh_ref,
                w0_ref, wout_ref, h_ref):
    t = pl.program_id(0)
    p = p_ref[...]
    pn = jnp.sqrt(jnp.sum(p * p)) + 1e-16
    vals = jnp.tanh(zt_ref[0] / pn)
    xh = xh_ref[0] * vals

    @pl.when(t == 0)
    def _():
        h_ref[...] = w0_ref[0]

    h = h_ref[...]
    gi = jnp.dot(xh, wih_ref[...], preferred_element_type=jnp.float32) + bih_ref[...]
    gh = jnp.dot(h, whh_ref[...], preferred_element_type=jnp.float32) + bhh_ref[...]
    r = jax.nn.sigmoid(gi[:, :F] + gh[:, :F])
    z = jax.nn.sigmoid(gi[:, F:2 * F] + gh[:, F:2 * F])
    nn = jnp.tanh(gi[:, 2 * F:] + r * gh[:, 2 * F:])
    h1 = (1.0 - z) * nn + z * h
    h_ref[...] = h1
    wout_ref[0] = h1


def _bigmm_body(a_ref, u_ref, di_ref, w_ref, o_ref):
    i = pl.program_id(1)
    a = a_ref[0].astype(jnp.float32)
    di = di_ref[0]
    acc = jnp.dot(a, u_ref[0], preferred_element_type=jnp.float32)
    mine = u_ref[0, pl.ds(i * BM, BM), :]
    o_ref[0] = di * jnp.dot(acc + mine, w_ref[0],
                            preferred_element_type=jnp.float32)


def _merge_sorted16(kk, vv, ck, cv):
    new_k, new_v = [], []
    for j in range(K8):
        rk = jax.lax.rev(ck, (0,))
        rv = jax.lax.rev(cv, (0,))
        m = kk[j] >= rk
        hik = jnp.where(m, kk[j], rk)
        hiv = jnp.where(m, vv[j], rv)
        lok = jnp.where(m, rk, kk[j])
        lov = jnp.where(m, rv, vv[j])
        hik, hiv = plsc.sort_key_val(hik, hiv, descending=True)
        ck, cv = plsc.sort_key_val(lok, lov, descending=True)
        new_k.append(hik)
        new_v.append(hiv)
    return tuple(new_k), tuple(new_v)


def _sc_topk_body(z_hbm, xflat_hbm, xh_hbm, ztop_hbm, z_v, idx_v, zk_v,
                  rows_v, sem):
    cid = jax.lax.axis_index("c")
    sid = jax.lax.axis_index("s")
    iota = jax.lax.iota(jnp.int32, L)
    for t in range(T):
        @pl.when((cid == t % 2) & (sid == t // 2))
        def _():
            pltpu.sync_copy(z_hbm.at[pl.ds(t * N, N)], z_v)
            kk0 = tuple(jnp.full((L,), -jnp.inf, jnp.float32)
                        for _ in range(K8))
            vv0 = tuple(jnp.zeros((L,), jnp.int32) for _ in range(K8))

            def step(j, carry):
                kk, vv = carry
                ck = z_v[pl.ds(j * L, L)]
                cv = iota + j * L
                sk, sv = plsc.sort_key_val(ck, cv, descending=True)
                return _merge_sorted16(kk, vv, sk, sv)

            kk, vv = jax.lax.fori_loop(0, N // L, step, (kk0, vv0))
            for j in range(K8):
                zk_v[pl.ds(j * L, L)] = kk[j]
                idx_v[pl.ds(j * L, L)] = vv[j] + t * N
            pltpu.async_copy(xflat_hbm.at[idx_v], rows_v, sem).wait()
            pltpu.sync_copy(rows_v, xh_hbm.at[pl.ds(t * F, F)])
            pltpu.sync_copy(zk_v, ztop_hbm.at[pl.ds(t * F, F)])


_sc_topk = functools.partial(
    pl.kernel,
    out_type=(jax.ShapeDtypeStruct((T * F, F), jnp.float32),
              jax.ShapeDtypeStruct((T * F,), jnp.float32)),
    mesh=plsc.VectorSubcoreMesh(core_axis_name="c", subcore_axis_name="s",
                                num_cores=2, num_subcores=16),
    scratch_types=(pltpu.VMEM((N,), jnp.float32),
                   pltpu.VMEM((F,), jnp.int32),
                   pltpu.VMEM((F,), jnp.float32),
                   pltpu.VMEM((F, F), jnp.float32),
                   pltpu.SemaphoreType.DMA),
    compiler_params=pltpu.CompilerParams(needs_layout_passes=False),
)(_sc_topk_body)


@jax.jit
def _impl(x_seq, A_seq, p, W_ih, W_hh, b_ih, b_hh, W0):
    xT_seq = jnp.transpose(x_seq, (0, 2, 1))
    p2 = p.reshape(1, F)
    W_ihT = W_ih.T
    W_hhT = W_hh.T
    b_ih2 = b_ih.reshape(1, 3 * F)
    b_hh2 = b_hh.reshape(1, 3 * F)

    z3 = pl.pallas_call(
        _score_body,
        grid=(T,),
        in_specs=[
            pl.BlockSpec((1, F, N), lambda t: (t, 0, 0)),
            pl.BlockSpec((1, F), lambda t: (0, 0)),
        ],
        out_specs=pl.BlockSpec((1, 1, N), lambda t: (t, 0, 0)),
        out_shape=jax.ShapeDtypeStruct((T, 1, N), jnp.float32),
        compiler_params=pltpu.CompilerParams(
            dimension_semantics=("arbitrary",)),
    )(xT_seq, p2)

    xh_flat, ztop = _sc_topk(z3.reshape(T * N), x_seq.reshape(T * N, F))
    xh_all = xh_flat.reshape(T, F, F)

    W_all = pl.pallas_call(
        _chain_body,
        grid=(T,),
        in_specs=[
            pl.BlockSpec((1, F, F), lambda t: (t, 0, 0)),
            pl.BlockSpec((1, F, 1), lambda t: (t, 0, 0)),
            pl.BlockSpec((1, F), lambda t: (0, 0)),
            pl.BlockSpec((F, 3 * F), lambda t: (0, 0)),
            pl.BlockSpec((F, 3 * F), lambda t: (0, 0)),
            pl.BlockSpec((1, 3 * F), lambda t: (0, 0)),
            pl.BlockSpec((1, 3 * F), lambda t: (0, 0)),
            pl.BlockSpec((1, F, F), lambda t: (0, 0, 0)),
        ],
        out_specs=pl.BlockSpec((1, F, F), lambda t: (t, 0, 0)),
        out_shape=jax.ShapeDtypeStruct((T, F, F), jnp.float32),
        scratch_shapes=[pltpu.VMEM((F, F), jnp.float32)],
        compiler_params=pltpu.CompilerParams(
            dimension_semantics=("arbitrary",)),
    )(xh_all, ztop.reshape(T, F, 1), p2, W_ihT, W_hhT, b_ih2, b_hh2, W0)

    u, dinv, A_bf = pl.pallas_call(
        _prep_body,
        grid=(T, N // BR),
        in_specs=[
            pl.BlockSpec((1, BR, N), lambda t, i: (t, i, 0)),
            pl.BlockSpec((1, BR, F), lambda t, i: (t, i, 0)),
        ],
        out_specs=[
            pl.BlockSpec((1, BR, F), lambda t, i: (t, i, 0)),
            pl.BlockSpec((1, BR, 1), lambda t, i: (t, i, 0)),
            pl.BlockSpec((1, BR, N), lambda t, i: (t, i, 0)),
        ],
        out_shape=[
            jax.ShapeDtypeStruct((T, N, F), jnp.float32),
            jax.ShapeDtypeStruct((T, N, 1), jnp.float32),
            jax.ShapeDtypeStruct((T, N, N), jnp.bfloat16),
        ],
        compiler_params=pltpu.CompilerParams(
            dimension_semantics=("arbitrary", "arbitrary")),
    )(A_seq, x_seq)

    out = pl.pallas_call(
        _bigmm_body,
        grid=(T, N // BM),
        in_specs=[
            pl.BlockSpec((1, BM, N), lambda t, i: (t, i, 0)),
            pl.BlockSpec((1, N, F), lambda t, i: (t, 0, 0)),
            pl.BlockSpec((1, BM, 1), lambda t, i: (t, i, 0)),
            pl.BlockSpec((1, F, F), lambda t, i: (t, 0, 0)),
        ],
        out_specs=pl.BlockSpec((1, BM, F), lambda t, i: (t, i, 0)),
        out_shape=jax.ShapeDtypeStruct((T, N, F), jnp.float32),
        compiler_params=pltpu.CompilerParams(
            dimension_semantics=("arbitrary", "arbitrary")),
    )(A_bf, u, dinv, W_all)
    return out


def kernel(x_seq, A_seq, p, W_ih, W_hh, b_ih, b_hh, W0):
    return _impl(x_seq, A_seq, p, W_ih, W_hh, b_ih, b_hh, W0)


# speedup vs baseline: 1.2843x; 1.2843x over previous
import functools

import jax
import jax.numpy as jnp
from jax.experimental import pallas as pl
from jax.experimental.pallas import tpu as pltpu
from jax.experimental.pallas import tpu_sc as plsc

T, N, F = 3, 4096, 128
BM = 512
NB = N // BM
L = 16
K8 = F // L


def _score_body(xt_ref, p_ref, z_ref):
    z_ref[0] = jax.lax.dot_general(
        p_ref[...], xt_ref[0], (((1,), (0,)), ((), ())),
        preferred_element_type=jnp.float32)


def _mega_body(a_ref, x_ref, w_ref, o_ref, res_ref, di_ref, u_ref, ub_ref):
    i = pl.program_id(1)

    @pl.when(i < NB)
    def _():
        a = a_ref[0]
        sm = jnp.sum(a, axis=1, keepdims=True) + 1.0
        di_ref[pl.ds(i * BM, BM), :] = jnp.where(sm > 0, jax.lax.rsqrt(sm), 0.0)
        res_ref[pl.ds(i * BM, BM), :] = a.astype(jnp.bfloat16)

    @pl.when(i == NB)
    def _():
        u = di_ref[...] * x_ref[0]
        u_ref[...] = u
        ub_ref[...] = u.astype(jnp.bfloat16)

    @pl.when(i >= NB)
    def _():
        b = i - NB
        ab = res_ref[pl.ds(b * BM, BM), :]
        acc = jnp.dot(ab, ub_ref[...], preferred_element_type=jnp.float32)
        ur = u_ref[pl.ds(b * BM, BM), :]
        dib = di_ref[pl.ds(b * BM, BM), :]
        o_ref[0] = dib * jnp.dot(acc + ur, w_ref[0],
                                 preferred_element_type=jnp.float32)


def _chain_body(xh_ref, zt_ref, p_ref, wih_ref, whh_ref, bih_ref, bhh_ref,
                w0_ref, wout_ref, h_ref):
    t = pl.program_id(0)
    p = p_ref[...]
    pn = jnp.sqrt(jnp.sum(p * p)) + 1e-16
    vals = jnp.tanh(zt_ref[0] / pn)
    xh = xh_ref[0] * vals

    @pl.when(t == 0)
    def _():
        h_ref[...] = w0_ref[0]

    h = h_ref[...]
    gi = jnp.dot(xh, wih_ref[...], preferred_element_type=jnp.float32) + bih_ref[...]
    gh = jnp.dot(h, whh_ref[...], preferred_element_type=jnp.float32) + bhh_ref[...]
    r = jax.nn.sigmoid(gi[:, :F] + gh[:, :F])
    z = jax.nn.sigmoid(gi[:, F:2 * F] + gh[:, F:2 * F])
    nn = jnp.tanh(gi[:, 2 * F:] + r * gh[:, 2 * F:])
    h1 = (1.0 - z) * nn + z * h
    h_ref[...] = h1
    wout_ref[0] = h1


def _merge_sorted16(kk, vv, ck, cv):
    new_k, new_v = [], []
    for j in range(K8):
        rk = jax.lax.rev(ck, (0,))
        rv = jax.lax.rev(cv, (0,))
        m = kk[j] >= rk
        hik = jnp.where(m, kk[j], rk)
        hiv = jnp.where(m, vv[j], rv)
        lok = jnp.where(m, rk, kk[j])
        lov = jnp.where(m, rv, vv[j])
        hik, hiv = plsc.sort_key_val(hik, hiv, descending=True)
        ck, cv = plsc.sort_key_val(lok, lov, descending=True)
        new_k.append(hik)
        new_v.append(hiv)
    return tuple(new_k), tuple(new_v)


def _sc_topk_body(z_hbm, xflat_hbm, xh_hbm, ztop_hbm, z_v, idx_v, zk_v,
                  rows_v, sem):
    cid = jax.lax.axis_index("c")
    sid = jax.lax.axis_index("s")
    iota = jax.lax.iota(jnp.int32, L)
    for t in range(T):
        @pl.when((cid == t % 2) & (sid == t // 2))
        def _():
            pltpu.sync_copy(z_hbm.at[pl.ds(t * N, N)], z_v)
            kk0 = tuple(jnp.full((L,), -jnp.inf, jnp.float32)
                        for _ in range(K8))
            vv0 = tuple(jnp.zeros((L,), jnp.int32) for _ in range(K8))

            def step(j, carry):
                kk, vv = carry
                ck = z_v[pl.ds(j * L, L)]
                cv = iota + j * L
                sk, sv = plsc.sort_key_val(ck, cv, descending=True)
                return _merge_sorted16(kk, vv, sk, sv)

            kk, vv = jax.lax.fori_loop(0, N // L, step, (kk0, vv0))
            for j in range(K8):
                zk_v[pl.ds(j * L, L)] = kk[j]
                idx_v[pl.ds(j * L, L)] = vv[j] + t * N
            pltpu.async_copy(xflat_hbm.at[idx_v], rows_v, sem).wait()
            pltpu.sync_copy(rows_v, xh_hbm.at[pl.ds(t * F, F)])
            pltpu.sync_copy(zk_v, ztop_hbm.at[pl.ds(t * F, F)])


_sc_topk = functools.partial(
    pl.kernel,
    out_type=(jax.ShapeDtypeStruct((T * F, F), jnp.float32),
              jax.ShapeDtypeStruct((T * F,), jnp.float32)),
    mesh=plsc.VectorSubcoreMesh(core_axis_name="c", subcore_axis_name="s",
                                num_cores=2, num_subcores=16),
    scratch_types=(pltpu.VMEM((N,), jnp.float32),
                   pltpu.VMEM((F,), jnp.int32),
                   pltpu.VMEM((F,), jnp.float32),
                   pltpu.VMEM((F, F), jnp.float32),
                   pltpu.SemaphoreType.DMA),
    compiler_params=pltpu.CompilerParams(needs_layout_passes=False),
)(_sc_topk_body)


@jax.jit
def _impl(x_seq, A_seq, p, W_ih, W_hh, b_ih, b_hh, W0):
    xT_seq = jnp.transpose(x_seq, (0, 2, 1))
    p2 = p.reshape(1, F)
    W_ihT = W_ih.T
    W_hhT = W_hh.T
    b_ih2 = b_ih.reshape(1, 3 * F)
    b_hh2 = b_hh.reshape(1, 3 * F)

    z3 = pl.pallas_call(
        _score_body,
        grid=(T,),
        in_specs=[
            pl.BlockSpec((1, F, N), lambda t: (t, 0, 0)),
            pl.BlockSpec((1, F), lambda t: (0, 0)),
        ],
        out_specs=pl.BlockSpec((1, 1, N), lambda t: (t, 0, 0)),
        out_shape=jax.ShapeDtypeStruct((T, 1, N), jnp.float32),
        compiler_params=pltpu.CompilerParams(
            dimension_semantics=("arbitrary",)),
    )(xT_seq, p2)

    xh_flat, ztop = _sc_topk(z3.reshape(T * N), x_seq.reshape(T * N, F))
    xh_all = xh_flat.reshape(T, F, F)

    W_all = pl.pallas_call(
        _chain_body,
        grid=(T,),
        in_specs=[
            pl.BlockSpec((1, F, F), lambda t: (t, 0, 0)),
            pl.BlockSpec((1, F, 1), lambda t: (t, 0, 0)),
            pl.BlockSpec((1, F), lambda t: (0, 0)),
            pl.BlockSpec((F, 3 * F), lambda t: (0, 0)),
            pl.BlockSpec((F, 3 * F), lambda t: (0, 0)),
            pl.BlockSpec((1, 3 * F), lambda t: (0, 0)),
            pl.BlockSpec((1, 3 * F), lambda t: (0, 0)),
            pl.BlockSpec((1, F, F), lambda t: (0, 0, 0)),
        ],
        out_specs=pl.BlockSpec((1, F, F), lambda t: (t, 0, 0)),
        out_shape=jax.ShapeDtypeStruct((T, F, F), jnp.float32),
        scratch_shapes=[pltpu.VMEM((F, F), jnp.float32)],
        compiler_params=pltpu.CompilerParams(
            dimension_semantics=("arbitrary",)),
    )(xh_all, ztop.reshape(T, F, 1), p2, W_ihT, W_hhT, b_ih2, b_hh2, W0)

    out = pl.pallas_call(
        _mega_body,
        grid=(T, 2 * NB),
        in_specs=[
            pl.BlockSpec((1, BM, N), lambda t, i: (t, jnp.minimum(i, NB - 1), 0)),
            pl.BlockSpec((1, N, F), lambda t, i: (t, 0, 0)),
            pl.BlockSpec((1, F, F), lambda t, i: (t, 0, 0)),
        ],
        out_specs=pl.BlockSpec(
            (1, BM, F), lambda t, i: (t, jnp.maximum(i - NB, 0), 0)),
        out_shape=jax.ShapeDtypeStruct((T, N, F), jnp.float32),
        scratch_shapes=[
            pltpu.VMEM((N, N), jnp.bfloat16),
            pltpu.VMEM((N, 1), jnp.float32),
            pltpu.VMEM((N, F), jnp.float32),
            pltpu.VMEM((N, F), jnp.bfloat16),
        ],
        compiler_params=pltpu.CompilerParams(
            dimension_semantics=("arbitrary", "arbitrary"),
            vmem_limit_bytes=100 * 1024 * 1024),
    )(A_seq, x_seq, W_all)
    return out


def kernel(x_seq, A_seq, p, W_ih, W_hh, b_ih, b_hh, W0):
    return _impl(x_seq, A_seq, p, W_ih, W_hh, b_ih, b_hh, W0)


# speedup vs baseline: 1.4663x; 1.1417x over previous
import functools

import jax
import jax.numpy as jnp
from jax.experimental import pallas as pl
from jax.experimental.pallas import tpu as pltpu
from jax.experimental.pallas import tpu_sc as plsc

T, N, F = 3, 4096, 128
BM = 512
NB = N // BM
L = 16
K8 = F // L


def _score_body(xt_ref, p_ref, z_ref):
    z_ref[0] = jax.lax.dot_general(
        p_ref[...], xt_ref[0], (((1,), (0,)), ((), ())),
        preferred_element_type=jnp.float32)


def _mega_body(a_ref, x_ref, w_ref, o_ref, res_ref, di_ref, u_ref, ub_ref):
    t = pl.program_id(0)
    i = pl.program_id(1)

    @pl.when((t > 0) & (i == 0))
    def _():
        di = di_ref[(t - 1) % 2]
        u = di * x_ref[0]
        u_ref[...] = u
        ub_ref[...] = u.astype(jnp.bfloat16)

    @pl.when(t > 0)
    def _():
        ab = res_ref[pl.ds(i * BM, BM), :]
        acc = jnp.dot(ab, ub_ref[...], preferred_element_type=jnp.float32)
        ur = u_ref[pl.ds(i * BM, BM), :]
        dib = di_ref[(t - 1) % 2, pl.ds(i * BM, BM), :]
        o_ref[0] = dib * jnp.dot(acc + ur, w_ref[0],
                                 preferred_element_type=jnp.float32)

    @pl.when(t < T)
    def _():
        a = a_ref[0]
        sm = jnp.sum(a, axis=1, keepdims=True) + 1.0
        di_ref[t % 2, pl.ds(i * BM, BM), :] = jnp.where(
            sm > 0, jax.lax.rsqrt(sm), 0.0)
        res_ref[pl.ds(i * BM, BM), :] = a.astype(jnp.bfloat16)


def _chain_body(xh_ref, zt_ref, p_ref, wih_ref, whh_ref, bih_ref, bhh_ref,
                w0_ref, wout_ref, h_ref):
    t = pl.program_id(0)
    p = p_ref[...]
    pn = jnp.sqrt(jnp.sum(p * p)) + 1e-16
    vals = jnp.tanh(zt_ref[0] / pn)
    xh = xh_ref[0] * vals

    @pl.when(t == 0)
    def _():
        h_ref[...] = w0_ref[0]

    h = h_ref[...]
    gi = jnp.dot(xh, wih_ref[...], preferred_element_type=jnp.float32) + bih_ref[...]
    gh = jnp.dot(h, whh_ref[...], preferred_element_type=jnp.float32) + bhh_ref[...]
    r = jax.nn.sigmoid(gi[:, :F] + gh[:, :F])
    z = jax.nn.sigmoid(gi[:, F:2 * F] + gh[:, F:2 * F])
    nn = jnp.tanh(gi[:, 2 * F:] + r * gh[:, 2 * F:])
    h1 = (1.0 - z) * nn + z * h
    h_ref[...] = h1
    wout_ref[0] = h1


def _merge_sorted16(kk, vv, ck, cv):
    new_k, new_v = [], []
    for j in range(K8):
        rk = jax.lax.rev(ck, (0,))
        rv = jax.lax.rev(cv, (0,))
        m = kk[j] >= rk
        hik = jnp.where(m, kk[j], rk)
        hiv = jnp.where(m, vv[j], rv)
        lok = jnp.where(m, rk, kk[j])
        lov = jnp.where(m, rv, vv[j])
        hik, hiv = plsc.sort_key_val(hik, hiv, descending=True)
        ck, cv = plsc.sort_key_val(lok, lov, descending=True)
        new_k.append(hik)
        new_v.append(hiv)
    return tuple(new_k), tuple(new_v)


def _sc_topk_body(z_hbm, xflat_hbm, xh_hbm, ztop_hbm, z_v, idx_v, zk_v,
                  rows_v, sem):
    cid = jax.lax.axis_index("c")
    sid = jax.lax.axis_index("s")
    iota = jax.lax.iota(jnp.int32, L)
    for t in range(T):
        @pl.when((cid == t % 2) & (sid == t // 2))
        def _():
            pltpu.sync_copy(z_hbm.at[pl.ds(t * N, N)], z_v)
            kk0 = tuple(jnp.full((L,), -jnp.inf, jnp.float32)
                        for _ in range(K8))
            vv0 = tuple(jnp.zeros((L,), jnp.int32) for _ in range(K8))

            def step(j, carry):
                kk, vv = carry
                ck = z_v[pl.ds(j * L, L)]
                cv = iota + j * L
                sk, sv = plsc.sort_key_val(ck, cv, descending=True)
                return _merge_sorted16(kk, vv, sk, sv)

            kk, vv = jax.lax.fori_loop(0, N // L, step, (kk0, vv0))
            for j in range(K8):
                zk_v[pl.ds(j * L, L)] = kk[j]
                idx_v[pl.ds(j * L, L)] = vv[j] + t * N
            pltpu.async_copy(xflat_hbm.at[idx_v], rows_v, sem).wait()
            pltpu.sync_copy(rows_v, xh_hbm.at[pl.ds(t * F, F)])
            pltpu.sync_copy(zk_v, ztop_hbm.at[pl.ds(t * F, F)])


_sc_topk = functools.partial(
    pl.kernel,
    out_type=(jax.ShapeDtypeStruct((T * F, F), jnp.float32),
              jax.ShapeDtypeStruct((T * F,), jnp.float32)),
    mesh=plsc.VectorSubcoreMesh(core_axis_name="c", subcore_axis_name="s",
                                num_cores=2, num_subcores=16),
    scratch_types=(pltpu.VMEM((N,), jnp.float32),
                   pltpu.VMEM((F,), jnp.int32),
                   pltpu.VMEM((F,), jnp.float32),
                   pltpu.VMEM((F, F), jnp.float32),
                   pltpu.SemaphoreType.DMA),
    compiler_params=pltpu.CompilerParams(needs_layout_passes=False),
)(_sc_topk_body)


@jax.jit
def _impl(x_seq, A_seq, p, W_ih, W_hh, b_ih, b_hh, W0):
    xT_seq = jnp.transpose(x_seq, (0, 2, 1))
    p2 = p.reshape(1, F)
    W_ihT = W_ih.T
    W_hhT = W_hh.T
    b_ih2 = b_ih.reshape(1, 3 * F)
    b_hh2 = b_hh.reshape(1, 3 * F)

    z3 = pl.pallas_call(
        _score_body,
        grid=(T,),
        in_specs=[
            pl.BlockSpec((1, F, N), lambda t: (t, 0, 0)),
            pl.BlockSpec((1, F), lambda t: (0, 0)),
        ],
        out_specs=pl.BlockSpec((1, 1, N), lambda t: (t, 0, 0)),
        out_shape=jax.ShapeDtypeStruct((T, 1, N), jnp.float32),
        compiler_params=pltpu.CompilerParams(
            dimension_semantics=("arbitrary",)),
    )(xT_seq, p2)

    xh_flat, ztop = _sc_topk(z3.reshape(T * N), x_seq.reshape(T * N, F))
    xh_all = xh_flat.reshape(T, F, F)

    W_all = pl.pallas_call(
        _chain_body,
        grid=(T,),
        in_specs=[
            pl.BlockSpec((1, F, F), lambda t: (t, 0, 0)),
            pl.BlockSpec((1, F, 1), lambda t: (t, 0, 0)),
            pl.BlockSpec((1, F), lambda t: (0, 0)),
            pl.BlockSpec((F, 3 * F), lambda t: (0, 0)),
            pl.BlockSpec((F, 3 * F), lambda t: (0, 0)),
            pl.BlockSpec((1, 3 * F), lambda t: (0, 0)),
            pl.BlockSpec((1, 3 * F), lambda t: (0, 0)),
            pl.BlockSpec((1, F, F), lambda t: (0, 0, 0)),
        ],
        out_specs=pl.BlockSpec((1, F, F), lambda t: (t, 0, 0)),
        out_shape=jax.ShapeDtypeStruct((T, F, F), jnp.float32),
        scratch_shapes=[pltpu.VMEM((F, F), jnp.float32)],
        compiler_params=pltpu.CompilerParams(
            dimension_semantics=("arbitrary",)),
    )(xh_all, ztop.reshape(T, F, 1), p2, W_ihT, W_hhT, b_ih2, b_hh2, W0)

    out = pl.pallas_call(
        _mega_body,
        grid=(T + 1, NB),
        in_specs=[
            pl.BlockSpec((1, BM, N), lambda t, i: (
                jnp.minimum(t, T - 1), jnp.where(t < T, i, NB - 1), 0)),
            pl.BlockSpec((1, N, F), lambda t, i: (jnp.maximum(t - 1, 0), 0, 0)),
            pl.BlockSpec((1, F, F), lambda t, i: (jnp.maximum(t - 1, 0), 0, 0)),
        ],
        out_specs=pl.BlockSpec((1, BM, F), lambda t, i: (
            jnp.maximum(t - 1, 0), jnp.where(t > 0, i, 0), 0)),
        out_shape=jax.ShapeDtypeStruct((T, N, F), jnp.float32),
        scratch_shapes=[
            pltpu.VMEM((N, N), jnp.bfloat16),
            pltpu.VMEM((2, N, 1), jnp.float32),
            pltpu.VMEM((N, F), jnp.float32),
            pltpu.VMEM((N, F), jnp.bfloat16),
        ],
        compiler_params=pltpu.CompilerParams(
            dimension_semantics=("arbitrary", "arbitrary"),
            vmem_limit_bytes=100 * 1024 * 1024),
    )(A_seq, x_seq, W_all)
    return out


def kernel(x_seq, A_seq, p, W_ih, W_hh, b_ih, b_hh, W0):
    return _impl(x_seq, A_seq, p, W_ih, W_hh, b_ih, b_hh, W0)
